# baseline (speedup 1.0000x reference)
import functools

import jax
import jax.numpy as jnp
import numpy as np
from jax.experimental import pallas as pl

_EPS = 0.05
_BLK = 2048


def _stop_kernel(x_ref, w1_ref, b1_ref, w2_ref, b2_ref, bits_ref, v_ref,
                 out_ref):
    x = x_ref[...]
    h = jnp.maximum(jnp.dot(x, w1_ref[...]) + b1_ref[...], 0.0)
    logits = jnp.dot(h, w2_ref[...]) + b2_ref[...]

    m = jnp.max(logits, axis=1, keepdims=True)
    e = jnp.exp(logits - m)
    p = e / jnp.sum(e, axis=1, keepdims=True)
    p = jnp.float32(1.0 - _EPS) * p + jnp.float32(_EPS * 0.05)
    lp = jnp.log(p)

    bits = bits_ref[...]
    fb = (bits >> jnp.uint32(9)) | jnp.uint32(0x3F800000)
    u = jax.lax.bitcast_convert_type(fb, jnp.float32) - jnp.float32(1.0)
    tiny = jnp.float32(np.finfo(np.float32).tiny)
    u = jnp.maximum(tiny, u * (jnp.float32(1.0) - tiny) + tiny)
    g = -jnp.log(-jnp.log(u))

    a = g + lp
    stop = (a[:, 0:1] >= a[:, 1:2]).astype(jnp.int32)
    out_ref[...] = stop * v_ref[0, 0]


@functools.partial(jax.jit, static_argnames=())
def kernel(x, t, halt_points, stop_W1, stop_b1, stop_W2, stop_b2,
           hop_W1, hop_b1, hop_W2, hop_b2,
           base_W1, base_b1, base_W2, base_b2):
    B, ninp = x.shape
    hid = stop_W1.shape[1]

    key = jax.random.key(42)
    k_act, _ = jax.random.split(key)
    bits = jax.random.bits(k_act, (B, 2), jnp.uint32)
    v = (jnp.float32(t) == jnp.float32(1.0)).astype(jnp.int32).reshape(1, 1)

    grid = (B // _BLK,)
    out = pl.pallas_call(
        _stop_kernel,
        grid=grid,
        in_specs=[
            pl.BlockSpec((_BLK, ninp), lambda i: (i, 0)),
            pl.BlockSpec((ninp, hid), lambda i: (0, 0)),
            pl.BlockSpec((1, hid), lambda i: (0, 0)),
            pl.BlockSpec((hid, 2), lambda i: (0, 0)),
            pl.BlockSpec((1, 2), lambda i: (0, 0)),
            pl.BlockSpec((_BLK, 2), lambda i: (i, 0)),
            pl.BlockSpec((1, 1), lambda i: (0, 0)),
        ],
        out_specs=pl.BlockSpec((_BLK, 1), lambda i: (i, 0)),
        out_shape=jax.ShapeDtypeStruct((B, 1), jnp.int32),
    )(x, stop_W1, stop_b1.reshape(1, hid), stop_W2, stop_b2.reshape(1, 2),
      bits, v)
    return out


# speedup vs baseline: 3.3108x; 3.3108x over previous
import functools

import jax
import jax.numpy as jnp
import numpy as np
from jax.experimental import pallas as pl

_EPS = 0.05
_BLK = 2048


def _stop_kernel(x_ref, w1_ref, b1_ref, w2_ref, b2_ref, bits_ref, v_ref,
                 out_ref):
    x = x_ref[...]
    h = jnp.maximum(jnp.dot(x, w1_ref[...]) + b1_ref[...], 0.0)
    lt = jax.lax.dot_general(w2_ref[...], h,
                             (((0,), (1,)), ((), ()))) + b2_ref[...]

    m = jnp.max(lt, axis=0, keepdims=True)
    e = jnp.exp(lt - m)
    p = e / jnp.sum(e, axis=0, keepdims=True)
    p = jnp.float32(1.0 - _EPS) * p + jnp.float32(_EPS * 0.05)
    lp = jnp.log(p)

    bits = bits_ref[...]
    fb = (bits >> jnp.uint32(9)) | jnp.uint32(0x3F800000)
    u = jax.lax.bitcast_convert_type(fb, jnp.float32) - jnp.float32(1.0)
    tiny = jnp.float32(np.finfo(np.float32).tiny)
    u = jnp.maximum(tiny, u * (jnp.float32(1.0) - tiny) + tiny)
    g = -jnp.log(-jnp.log(u))

    a = g + lp
    stop = (a[0:1, :] >= a[1:2, :]).astype(jnp.int32) * v_ref[0, 0]
    out_ref[...] = stop[None]


@functools.partial(jax.jit, static_argnames=())
def kernel(x, t, halt_points, stop_W1, stop_b1, stop_W2, stop_b2,
           hop_W1, hop_b1, hop_W2, hop_b2,
           base_W1, base_b1, base_W2, base_b2):
    B, ninp = x.shape
    hid = stop_W1.shape[1]
    nblk = B // _BLK

    key = jax.random.key(42)
    k_act, _ = jax.random.split(key)
    bits_t = jax.random.bits(k_act, (B, 2), jnp.uint32).T
    v = (jnp.float32(t) == jnp.float32(1.0)).astype(jnp.int32).reshape(1, 1)

    out = pl.pallas_call(
        _stop_kernel,
        grid=(nblk,),
        in_specs=[
            pl.BlockSpec((_BLK, ninp), lambda i: (i, 0)),
            pl.BlockSpec((ninp, hid), lambda i: (0, 0)),
            pl.BlockSpec((1, hid), lambda i: (0, 0)),
            pl.BlockSpec((hid, 2), lambda i: (0, 0)),
            pl.BlockSpec((2, 1), lambda i: (0, 0)),
            pl.BlockSpec((2, _BLK), lambda i: (0, i)),
            pl.BlockSpec((1, 1), lambda i: (0, 0)),
        ],
        out_specs=pl.BlockSpec((1, 1, _BLK), lambda i: (i, 0, 0)),
        out_shape=jax.ShapeDtypeStruct((nblk, 1, _BLK), jnp.int32),
    )(x, stop_W1, stop_b1.reshape(1, hid), stop_W2, stop_b2.reshape(2, 1),
      bits_t, v)
    return out.reshape(B, 1)


# speedup vs baseline: 4.8345x; 1.4602x over previous
import functools

import jax
import jax.numpy as jnp
import numpy as np
from jax.experimental import pallas as pl

_EPS = 0.05
_BLK = 2048
_ROTS = ((13, 15, 26, 6), (17, 29, 16, 24))


def _np_threefry2x32(ks0, ks1, x0, x1):
    ks2 = np.uint32(ks0 ^ ks1 ^ np.uint32(0x1BD11BDA))
    ks = [np.uint32(ks0), np.uint32(ks1), ks2]
    x0 = (np.asarray(x0, np.uint32) + ks[0]).astype(np.uint32)
    x1 = (np.asarray(x1, np.uint32) + ks[1]).astype(np.uint32)
    for i in range(5):
        for r in _ROTS[i % 2]:
            x0 = (x0 + x1).astype(np.uint32)
            x1 = ((x1 << np.uint32(r)) | (x1 >> np.uint32(32 - r))).astype(np.uint32)
            x1 = (x1 ^ x0).astype(np.uint32)
        x0 = (x0 + ks[(i + 1) % 3]).astype(np.uint32)
        x1 = (x1 + ks[(i + 2) % 3] + np.uint32(i + 1)).astype(np.uint32)
    return x0, x1


def _derive_act_key(seed):
    kd0, kd1 = np.uint32(0), np.uint32(seed)
    b1, b2 = _np_threefry2x32(kd0, kd1, np.zeros(2, np.uint32),
                              np.arange(2, dtype=np.uint32))
    return int(b1[0]), int(b2[0])


_KS0, _KS1 = _derive_act_key(42)


def _stop_kernel(x_ref, w1_ref, b1_ref, w2_ref, b2_ref, v_ref, out_ref):
    x = x_ref[...]
    h = jnp.maximum(jnp.dot(x, w1_ref[...]) + b1_ref[...], 0.0)
    lt = jax.lax.dot_general(w2_ref[...], h,
                             (((0,), (1,)), ((), ()))) + b2_ref[...]

    m = jnp.max(lt, axis=0, keepdims=True)
    e = jnp.exp(lt - m)
    p = e / jnp.sum(e, axis=0, keepdims=True)
    p = jnp.float32(1.0 - _EPS) * p + jnp.float32(_EPS * 0.05)
    lp = jnp.log(p)

    blk = out_ref.shape[2]
    base = jnp.uint32(pl.program_id(0) * blk)
    j = jax.lax.broadcasted_iota(jnp.uint32, (2, blk), 1)
    c = jax.lax.broadcasted_iota(jnp.uint32, (2, blk), 0)
    n = jnp.uint32(2) * (base + j) + c
    ks0 = jnp.uint32(_KS0)
    ks1 = jnp.uint32(_KS1)
    ks2 = jnp.uint32(_KS0 ^ _KS1 ^ 0x1BD11BDA)
    ks = (ks0, ks1, ks2)
    x0 = jnp.zeros((2, blk), jnp.uint32) + ks0
    x1 = n + ks1
    for i in range(5):
        for r in _ROTS[i % 2]:
            x0 = x0 + x1
            x1 = (x1 << jnp.uint32(r)) | (x1 >> jnp.uint32(32 - r))
            x1 = x1 ^ x0
        x0 = x0 + ks[(i + 1) % 3]
        x1 = x1 + ks[(i + 2) % 3] + jnp.uint32(i + 1)
    word = x0 ^ x1

    fb = (word >> jnp.uint32(9)) | jnp.uint32(0x3F800000)
    u = jax.lax.bitcast_convert_type(fb, jnp.float32) - jnp.float32(1.0)
    tiny = jnp.float32(np.finfo(np.float32).tiny)
    u = jnp.maximum(tiny, u * (jnp.float32(1.0) - tiny) + tiny)
    g = -jnp.log(-jnp.log(u))

    a = g + lp
    stop = (a[0:1, :] >= a[1:2, :]).astype(jnp.int32) * v_ref[0, 0]
    out_ref[...] = stop[None]


@functools.partial(jax.jit, static_argnames=())
def kernel(x, t, halt_points, stop_W1, stop_b1, stop_W2, stop_b2,
           hop_W1, hop_b1, hop_W2, hop_b2,
           base_W1, base_b1, base_W2, base_b2):
    B, ninp = x.shape
    hid = stop_W1.shape[1]
    nblk = B // _BLK

    v = (jnp.float32(t) == jnp.float32(1.0)).astype(jnp.int32).reshape(1, 1)

    out = pl.pallas_call(
        _stop_kernel,
        grid=(nblk,),
        in_specs=[
            pl.BlockSpec((_BLK, ninp), lambda i: (i, 0)),
            pl.BlockSpec((ninp, hid), lambda i: (0, 0)),
            pl.BlockSpec((1, hid), lambda i: (0, 0)),
            pl.BlockSpec((hid, 2), lambda i: (0, 0)),
            pl.BlockSpec((2, 1), lambda i: (0, 0)),
            pl.BlockSpec((1, 1), lambda i: (0, 0)),
        ],
        out_specs=pl.BlockSpec((1, 1, _BLK), lambda i: (i, 0, 0)),
        out_shape=jax.ShapeDtypeStruct((nblk, 1, _BLK), jnp.int32),
    )(x, stop_W1, stop_b1.reshape(1, hid), stop_W2, stop_b2.reshape(2, 1), v)
    return out.reshape(B, 1)


# speedup vs baseline: 4.8397x; 1.0011x over previous
import functools

import jax
import jax.numpy as jnp
import numpy as np
from jax.experimental import pallas as pl

_EPS = 0.05
_BLK = 2048
_ROTS = ((13, 15, 26, 6), (17, 29, 16, 24))


def _np_threefry2x32(ks0, ks1, x0, x1):
    ks2 = np.uint32(ks0 ^ ks1 ^ np.uint32(0x1BD11BDA))
    ks = [np.uint32(ks0), np.uint32(ks1), ks2]
    x0 = (np.asarray(x0, np.uint32) + ks[0]).astype(np.uint32)
    x1 = (np.asarray(x1, np.uint32) + ks[1]).astype(np.uint32)
    for i in range(5):
        for r in _ROTS[i % 2]:
            x0 = (x0 + x1).astype(np.uint32)
            x1 = ((x1 << np.uint32(r)) | (x1 >> np.uint32(32 - r))).astype(np.uint32)
            x1 = (x1 ^ x0).astype(np.uint32)
        x0 = (x0 + ks[(i + 1) % 3]).astype(np.uint32)
        x1 = (x1 + ks[(i + 2) % 3] + np.uint32(i + 1)).astype(np.uint32)
    return x0, x1


def _derive_act_key(seed):
    kd0, kd1 = np.uint32(0), np.uint32(seed)
    b1, b2 = _np_threefry2x32(kd0, kd1, np.zeros(2, np.uint32),
                              np.arange(2, dtype=np.uint32))
    return int(b1[0]), int(b2[0])


_KS0, _KS1 = _derive_act_key(42)


def _stop_kernel(x_ref, w1_ref, b1_ref, w2_ref, b2_ref, t_ref, out_ref):
    x = x_ref[...]
    h = jnp.maximum(jnp.dot(x, w1_ref[...]) + b1_ref[...], 0.0)
    lt = jax.lax.dot_general(w2_ref[...], h,
                             (((0,), (1,)), ((), ()))) + b2_ref[...]

    m = jnp.max(lt, axis=0, keepdims=True)
    e = jnp.exp(lt - m)
    p = e / jnp.sum(e, axis=0, keepdims=True)
    p = jnp.float32(1.0 - _EPS) * p + jnp.float32(_EPS * 0.05)
    lp = jnp.log(p)

    blk = out_ref.shape[2]
    base = jnp.uint32(pl.program_id(0) * blk)
    j = jax.lax.broadcasted_iota(jnp.uint32, (2, blk), 1)
    c = jax.lax.broadcasted_iota(jnp.uint32, (2, blk), 0)
    n = jnp.uint32(2) * (base + j) + c
    ks0 = jnp.uint32(_KS0)
    ks1 = jnp.uint32(_KS1)
    ks2 = jnp.uint32(_KS0 ^ _KS1 ^ 0x1BD11BDA)
    ks = (ks0, ks1, ks2)
    x0 = jnp.zeros((2, blk), jnp.uint32) + ks0
    x1 = n + ks1
    for i in range(5):
        for r in _ROTS[i % 2]:
            x0 = x0 + x1
            x1 = (x1 << jnp.uint32(r)) | (x1 >> jnp.uint32(32 - r))
            x1 = x1 ^ x0
        x0 = x0 + ks[(i + 1) % 3]
        x1 = x1 + ks[(i + 2) % 3] + jnp.uint32(i + 1)
    word = x0 ^ x1

    fb = (word >> jnp.uint32(9)) | jnp.uint32(0x3F800000)
    u = jax.lax.bitcast_convert_type(fb, jnp.float32) - jnp.float32(1.0)
    tiny = jnp.float32(np.finfo(np.float32).tiny)
    u = jnp.maximum(tiny, u * (jnp.float32(1.0) - tiny) + tiny)
    g = -jnp.log(-jnp.log(u))

    a = g + lp
    v = (t_ref[0, 0] == 1).astype(jnp.int32)
    stop = (a[0:1, :] >= a[1:2, :]).astype(jnp.int32) * v
    out_ref[...] = stop[None]


@functools.partial(jax.jit, static_argnames=())
def kernel(x, t, halt_points, stop_W1, stop_b1, stop_W2, stop_b2,
           hop_W1, hop_b1, hop_W2, hop_b2,
           base_W1, base_b1, base_W2, base_b2):
    B, ninp = x.shape
    hid = stop_W1.shape[1]
    nblk = B // _BLK

    t_arr = jnp.asarray(t, jnp.int32).reshape(1, 1)

    out = pl.pallas_call(
        _stop_kernel,
        grid=(nblk,),
        in_specs=[
            pl.BlockSpec((_BLK, ninp), lambda i: (i, 0)),
            pl.BlockSpec((ninp, hid), lambda i: (0, 0)),
            pl.BlockSpec((1, hid), lambda i: (0, 0)),
            pl.BlockSpec((hid, 2), lambda i: (0, 0)),
            pl.BlockSpec((2, 1), lambda i: (0, 0)),
            pl.BlockSpec((1, 1), lambda i: (0, 0)),
        ],
        out_specs=pl.BlockSpec((1, 1, _BLK), lambda i: (i, 0, 0)),
        out_shape=jax.ShapeDtypeStruct((nblk, 1, _BLK), jnp.int32),
    )(x, stop_W1, stop_b1.reshape(1, hid), stop_W2, stop_b2.reshape(2, 1),
      t_arr)
    return out.reshape(B, 1)


# speedup vs baseline: 5.5720x; 1.1513x over previous
import functools

import jax
import jax.numpy as jnp
import numpy as np
from jax.experimental import pallas as pl

_EPS = 0.05
_BLK = 4096
_ROTS = ((13, 15, 26, 6), (17, 29, 16, 24))


def _np_threefry2x32(ks0, ks1, x0, x1):
    ks2 = np.uint32(ks0 ^ ks1 ^ np.uint32(0x1BD11BDA))
    ks = [np.uint32(ks0), np.uint32(ks1), ks2]
    x0 = (np.asarray(x0, np.uint32) + ks[0]).astype(np.uint32)
    x1 = (np.asarray(x1, np.uint32) + ks[1]).astype(np.uint32)
    for i in range(5):
        for r in _ROTS[i % 2]:
            x0 = (x0 + x1).astype(np.uint32)
            x1 = ((x1 << np.uint32(r)) | (x1 >> np.uint32(32 - r))).astype(np.uint32)
            x1 = (x1 ^ x0).astype(np.uint32)
        x0 = (x0 + ks[(i + 1) % 3]).astype(np.uint32)
        x1 = (x1 + ks[(i + 2) % 3] + np.uint32(i + 1)).astype(np.uint32)
    return x0, x1


def _derive_act_key(seed):
    kd0, kd1 = np.uint32(0), np.uint32(seed)
    b1, b2 = _np_threefry2x32(kd0, kd1, np.zeros(2, np.uint32),
                              np.arange(2, dtype=np.uint32))
    return int(b1[0]), int(b2[0])


_KS0, _KS1 = _derive_act_key(42)


def _stop_kernel(x_ref, w1_ref, b1_ref, w2_ref, b2_ref, t_ref, out_ref):
    x = x_ref[...]
    h = jnp.maximum(jnp.dot(x, w1_ref[...]) + b1_ref[...], 0.0)
    lt = jax.lax.dot_general(w2_ref[...], h,
                             (((0,), (1,)), ((), ()))) + b2_ref[...]

    m = jnp.max(lt, axis=0, keepdims=True)
    e = jnp.exp(lt - m)
    p = e / jnp.sum(e, axis=0, keepdims=True)
    p = jnp.float32(1.0 - _EPS) * p + jnp.float32(_EPS * 0.05)
    lp = jnp.log(p)

    blk = out_ref.shape[2]
    base = jnp.uint32(pl.program_id(0) * blk)
    j = jax.lax.broadcasted_iota(jnp.uint32, (2, blk), 1)
    c = jax.lax.broadcasted_iota(jnp.uint32, (2, blk), 0)
    n = jnp.uint32(2) * (base + j) + c
    ks0 = jnp.uint32(_KS0)
    ks1 = jnp.uint32(_KS1)
    ks2 = jnp.uint32(_KS0 ^ _KS1 ^ 0x1BD11BDA)
    ks = (ks0, ks1, ks2)
    x0 = jnp.zeros((2, blk), jnp.uint32) + ks0
    x1 = n + ks1
    for i in range(5):
        for r in _ROTS[i % 2]:
            x0 = x0 + x1
            x1 = (x1 << jnp.uint32(r)) | (x1 >> jnp.uint32(32 - r))
            x1 = x1 ^ x0
        x0 = x0 + ks[(i + 1) % 3]
        x1 = x1 + ks[(i + 2) % 3] + jnp.uint32(i + 1)
    word = x0 ^ x1

    fb = (word >> jnp.uint32(9)) | jnp.uint32(0x3F800000)
    u = jax.lax.bitcast_convert_type(fb, jnp.float32) - jnp.float32(1.0)
    tiny = jnp.float32(np.finfo(np.float32).tiny)
    u = jnp.maximum(tiny, u * (jnp.float32(1.0) - tiny) + tiny)
    g = -jnp.log(-jnp.log(u))

    a = g + lp
    v = (t_ref[0, 0] == 1).astype(jnp.int32)
    stop = (a[0:1, :] >= a[1:2, :]).astype(jnp.int32) * v
    out_ref[...] = stop[None]


@functools.partial(jax.jit, static_argnames=())
def kernel(x, t, halt_points, stop_W1, stop_b1, stop_W2, stop_b2,
           hop_W1, hop_b1, hop_W2, hop_b2,
           base_W1, base_b1, base_W2, base_b2):
    B, ninp = x.shape
    hid = stop_W1.shape[1]
    nblk = B // _BLK

    t_arr = jnp.asarray(t, jnp.int32).reshape(1, 1)

    out = pl.pallas_call(
        _stop_kernel,
        grid=(nblk,),
        in_specs=[
            pl.BlockSpec((_BLK, ninp), lambda i: (i, 0)),
            pl.BlockSpec((ninp, hid), lambda i: (0, 0)),
            pl.BlockSpec((1, hid), lambda i: (0, 0)),
            pl.BlockSpec((hid, 2), lambda i: (0, 0)),
            pl.BlockSpec((2, 1), lambda i: (0, 0)),
            pl.BlockSpec((1, 1), lambda i: (0, 0)),
        ],
        out_specs=pl.BlockSpec((1, 1, _BLK), lambda i: (i, 0, 0)),
        out_shape=jax.ShapeDtypeStruct((nblk, 1, _BLK), jnp.int32),
    )(x, stop_W1, stop_b1.reshape(1, hid), stop_W2, stop_b2.reshape(2, 1),
      t_arr)
    return out.reshape(B, 1)


# speedup vs baseline: 5.7760x; 1.0366x over previous
import functools

import jax
import jax.numpy as jnp
import numpy as np
from jax.experimental import pallas as pl

_EPS = 0.05
_BLK = 8192
_ROTS = ((13, 15, 26, 6), (17, 29, 16, 24))


def _np_threefry2x32(ks0, ks1, x0, x1):
    ks2 = np.uint32(ks0 ^ ks1 ^ np.uint32(0x1BD11BDA))
    ks = [np.uint32(ks0), np.uint32(ks1), ks2]
    x0 = (np.asarray(x0, np.uint32) + ks[0]).astype(np.uint32)
    x1 = (np.asarray(x1, np.uint32) + ks[1]).astype(np.uint32)
    for i in range(5):
        for r in _ROTS[i % 2]:
            x0 = (x0 + x1).astype(np.uint32)
            x1 = ((x1 << np.uint32(r)) | (x1 >> np.uint32(32 - r))).astype(np.uint32)
            x1 = (x1 ^ x0).astype(np.uint32)
        x0 = (x0 + ks[(i + 1) % 3]).astype(np.uint32)
        x1 = (x1 + ks[(i + 2) % 3] + np.uint32(i + 1)).astype(np.uint32)
    return x0, x1


def _derive_act_key(seed):
    kd0, kd1 = np.uint32(0), np.uint32(seed)
    b1, b2 = _np_threefry2x32(kd0, kd1, np.zeros(2, np.uint32),
                              np.arange(2, dtype=np.uint32))
    return int(b1[0]), int(b2[0])


_KS0, _KS1 = _derive_act_key(42)


def _stop_kernel(x_ref, w1_ref, b1_ref, w2_ref, b2_ref, t_ref, out_ref):
    x = x_ref[...]
    h = jnp.maximum(jnp.dot(x, w1_ref[...]) + b1_ref[...], 0.0)
    lt = jax.lax.dot_general(w2_ref[...], h,
                             (((0,), (1,)), ((), ()))) + b2_ref[...]

    m = jnp.max(lt, axis=0, keepdims=True)
    e = jnp.exp(lt - m)
    p = e / jnp.sum(e, axis=0, keepdims=True)
    p = jnp.float32(1.0 - _EPS) * p + jnp.float32(_EPS * 0.05)
    lp = jnp.log(p)

    blk = out_ref.shape[2]
    base = jnp.uint32(pl.program_id(0) * blk)
    j = jax.lax.broadcasted_iota(jnp.uint32, (2, blk), 1)
    c = jax.lax.broadcasted_iota(jnp.uint32, (2, blk), 0)
    n = jnp.uint32(2) * (base + j) + c
    ks0 = jnp.uint32(_KS0)
    ks1 = jnp.uint32(_KS1)
    ks2 = jnp.uint32(_KS0 ^ _KS1 ^ 0x1BD11BDA)
    ks = (ks0, ks1, ks2)
    x0 = jnp.zeros((2, blk), jnp.uint32) + ks0
    x1 = n + ks1
    for i in range(5):
        for r in _ROTS[i % 2]:
            x0 = x0 + x1
            x1 = (x1 << jnp.uint32(r)) | (x1 >> jnp.uint32(32 - r))
            x1 = x1 ^ x0
        x0 = x0 + ks[(i + 1) % 3]
        x1 = x1 + ks[(i + 2) % 3] + jnp.uint32(i + 1)
    word = x0 ^ x1

    fb = (word >> jnp.uint32(9)) | jnp.uint32(0x3F800000)
    u = jax.lax.bitcast_convert_type(fb, jnp.float32) - jnp.float32(1.0)
    tiny = jnp.float32(np.finfo(np.float32).tiny)
    u = jnp.maximum(tiny, u * (jnp.float32(1.0) - tiny) + tiny)
    g = -jnp.log(-jnp.log(u))

    a = g + lp
    v = (t_ref[0, 0] == 1).astype(jnp.int32)
    stop = (a[0:1, :] >= a[1:2, :]).astype(jnp.int32) * v
    out_ref[...] = stop[None]


@functools.partial(jax.jit, static_argnames=())
def kernel(x, t, halt_points, stop_W1, stop_b1, stop_W2, stop_b2,
           hop_W1, hop_b1, hop_W2, hop_b2,
           base_W1, base_b1, base_W2, base_b2):
    B, ninp = x.shape
    hid = stop_W1.shape[1]
    nblk = B // _BLK

    t_arr = jnp.asarray(t, jnp.int32).reshape(1, 1)

    out = pl.pallas_call(
        _stop_kernel,
        grid=(nblk,),
        in_specs=[
            pl.BlockSpec((_BLK, ninp), lambda i: (i, 0)),
            pl.BlockSpec((ninp, hid), lambda i: (0, 0)),
            pl.BlockSpec((1, hid), lambda i: (0, 0)),
            pl.BlockSpec((hid, 2), lambda i: (0, 0)),
            pl.BlockSpec((2, 1), lambda i: (0, 0)),
            pl.BlockSpec((1, 1), lambda i: (0, 0)),
        ],
        out_specs=pl.BlockSpec((1, 1, _BLK), lambda i: (i, 0, 0)),
        out_shape=jax.ShapeDtypeStruct((nblk, 1, _BLK), jnp.int32),
    )(x, stop_W1, stop_b1.reshape(1, hid), stop_W2, stop_b2.reshape(2, 1),
      t_arr)
    return out.reshape(B, 1)
